# Initial kernel scaffold; baseline (speedup 1.0000x reference)
import jax
import jax.numpy as jnp
from jax.experimental import pallas as pl


def kernel(hidden_states, gate_w, expert_bias, w1, w3, w2, sw1, sw3, sw2):
    raise NotImplementedError("write your pallas kernel here")


# speedup vs baseline: 2.0831x; 2.0831x over previous
import jax
import jax.numpy as jnp
from jax.experimental import pallas as pl
from jax.experimental.pallas import tpu as pltpu

_T, _D, _E, _TOPK, _NG, _TG, _DFF, _DFFS = 2048, 1024, 16, 2, 4, 2, 512, 512
_GS = _E // _NG
_ROUTE_SCALE = 2.5


def _router_body(x_ref, gw_ref, eb_ref, comb_ref):
    x = x_ref[...]
    logits = jax.lax.dot_general(
        x, gw_ref[...], (((1,), (1,)), ((), ())),
        preferred_element_type=jnp.float32)
    scores = jax.nn.sigmoid(logits)
    sfc = scores + eb_ref[...]
    gs_cols = []
    for g in range(_NG):
        c = [sfc[:, g * _GS + i:g * _GS + i + 1] for i in range(_GS)]
        best = None
        for i in range(_GS):
            for j in range(i + 1, _GS):
                s = c[i] + c[j]
                best = s if best is None else jnp.maximum(best, s)
        gs_cols.append(best)
    gs = jnp.concatenate(gs_cols, axis=1)
    gidx = jax.lax.broadcasted_iota(jnp.int32, (_T, _NG), 1)
    grank = jnp.zeros((_T, _NG), jnp.float32)
    for j in range(_NG):
        gj = gs[:, j:j + 1]
        grank += jnp.where((gj > gs) | ((gj == gs) & (j < gidx)), 1.0, 0.0)
    gsel = (grank < _TG).astype(jnp.float32)
    emask = jnp.concatenate(
        [gsel[:, e // _GS:e // _GS + 1] for e in range(_E)], axis=1)
    tmp = sfc * emask
    eidx = jax.lax.broadcasted_iota(jnp.int32, (_T, _E), 1)
    erank = jnp.zeros((_T, _E), jnp.float32)
    for j in range(_E):
        vj = tmp[:, j:j + 1]
        erank += jnp.where((vj > tmp) | ((vj == tmp) & (j < eidx)), 1.0, 0.0)
    sel = jnp.where(erank < _TOPK, 1.0, 0.0)
    w = scores * sel
    denom = jnp.sum(w, axis=1, keepdims=True) + 1e-20
    comb_ref[...] = w * (_ROUTE_SCALE / denom)


def _dot_t(a, b):
    return jax.lax.dot_general(
        a, b, (((1,), (1,)), ((), ())), preferred_element_type=jnp.float32)


def _moe_body(xb_ref, comb_ref, w1_ref, w3_ref, w2_ref,
              sw1_ref, sw3_ref, sw2_ref, out_ref):
    e = pl.program_id(0)
    xb = xb_ref[...]

    @pl.when(e == 0)
    def _init():
        g0 = _dot_t(xb, sw1_ref[...].astype(jnp.bfloat16))
        u0 = _dot_t(xb, sw3_ref[...].astype(jnp.bfloat16))
        h0 = (g0 * jax.nn.sigmoid(g0) * u0).astype(jnp.bfloat16)
        out_ref[...] = _dot_t(h0, sw2_ref[...].astype(jnp.bfloat16))

    g = _dot_t(xb, w1_ref[0].astype(jnp.bfloat16))
    u = _dot_t(xb, w3_ref[0].astype(jnp.bfloat16))
    h = (g * jax.nn.sigmoid(g) * u).astype(jnp.bfloat16)
    y = _dot_t(h, w2_ref[0].astype(jnp.bfloat16))
    onehot = (jax.lax.broadcasted_iota(jnp.int32, (_E, 1), 0) == e
              ).astype(jnp.float32)
    c = jax.lax.dot_general(
        comb_ref[...], onehot, (((1,), (0,)), ((), ())),
        preferred_element_type=jnp.float32)
    out_ref[...] += y * c


def kernel(hidden_states, gate_w, expert_bias, w1, w3, w2, sw1, sw3, sw2):
    x = hidden_states.reshape(_T, _D)
    eb = expert_bias.reshape(1, _E)
    comb = pl.pallas_call(
        _router_body,
        out_shape=jax.ShapeDtypeStruct((_T, _E), jnp.float32),
    )(x, gate_w, eb)
    xb = x.astype(jnp.bfloat16)
    out = pl.pallas_call(
        _moe_body,
        grid=(_E,),
        in_specs=[
            pl.BlockSpec((_T, _D), lambda e: (0, 0)),
            pl.BlockSpec((_T, _E), lambda e: (0, 0)),
            pl.BlockSpec((1, _DFF, _D), lambda e: (e, 0, 0)),
            pl.BlockSpec((1, _DFF, _D), lambda e: (e, 0, 0)),
            pl.BlockSpec((1, _D, _DFF), lambda e: (e, 0, 0)),
            pl.BlockSpec((_DFFS, _D), lambda e: (0, 0)),
            pl.BlockSpec((_DFFS, _D), lambda e: (0, 0)),
            pl.BlockSpec((_D, _DFFS), lambda e: (0, 0)),
        ],
        out_specs=pl.BlockSpec((_T, _D), lambda e: (0, 0)),
        out_shape=jax.ShapeDtypeStruct((_T, _D), jnp.float32),
    )(xb, comb, w1, w3, w2, sw1, sw3, sw2)
    return out
